# baseline (speedup 1.0000x reference)
import functools

import jax
import jax.numpy as jnp
from jax import lax
from jax.experimental import pallas as pl
from jax.experimental.pallas import tpu as pltpu

_NUM_FEATURES = 128
_ZMAX = 87
_TPAD = 128
_BLK = 1024


def _table_body(emb_ref, ec_ref, cw_ref, out_ref):
    out_ref[...] = emb_ref[...] + lax.dot_general(
        ec_ref[...], cw_ref[...],
        dimension_numbers=(((1,), (1,)), ((), ())),
        preferred_element_type=jnp.float32,
    )


def _build_table_padded(element_embedding, config_weight, electron_config):
    emb_p = jnp.zeros((_TPAD, _NUM_FEATURES), jnp.float32).at[:_ZMAX].set(
        element_embedding)
    ec_p = jnp.zeros((_TPAD, electron_config.shape[1]), jnp.float32).at[
        :_ZMAX].set(electron_config)
    return pl.pallas_call(
        _table_body,
        out_shape=jax.ShapeDtypeStruct((_TPAD, _NUM_FEATURES), jnp.float32),
    )(emb_p, ec_p, config_weight)


def _gather_body(z_ref, tab_ref, out_ref):
    z = z_ref[0]
    iota = lax.broadcasted_iota(jnp.int32, (1, _TPAD), 1)
    onehot = (z == iota).astype(jnp.float32)
    out_ref[...] = jnp.dot(onehot, tab_ref[...],
                           preferred_element_type=jnp.float32)


def _tc_gather(table_p, z_flat):
    n = z_flat.shape[0]
    nblk = n // _BLK
    zr = z_flat.reshape(nblk, _BLK, 1)
    return pl.pallas_call(
        _gather_body,
        grid=(nblk,),
        in_specs=[
            pl.BlockSpec((1, _BLK, 1), lambda i: (i, 0, 0)),
            pl.BlockSpec((_TPAD, _NUM_FEATURES), lambda i: (0, 0)),
        ],
        out_specs=pl.BlockSpec((_BLK, _NUM_FEATURES), lambda i: (i, 0)),
        out_shape=jax.ShapeDtypeStruct((n, _NUM_FEATURES), jnp.float32),
    )(zr, table_p)


def kernel(Z, element_embedding, config_weight, electron_config):
    table_p = _build_table_padded(element_embedding, config_weight,
                                  electron_config)
    out = _tc_gather(table_p, Z.reshape(-1))
    return out.reshape(Z.shape + (_NUM_FEATURES,))


# speedup vs baseline: 4.7015x; 4.7015x over previous
import functools

import jax
import jax.numpy as jnp
from jax import lax
from jax.experimental import pallas as pl
from jax.experimental.pallas import tpu as pltpu
from jax.experimental.pallas import tpu_sc as plsc

_NUM_FEATURES = 128
_ZMAX = 87

_NUM_CORES = 2
_NUM_SUBCORES = 16
_NW = _NUM_CORES * _NUM_SUBCORES

_CHUNK = 64
_NBUF = 6
_LOOK = 3


def _table_body(emb_ref, ec_ref, cw_ref, out_ref):
    out_ref[...] = emb_ref[...] + lax.dot_general(
        ec_ref[...], cw_ref[...],
        dimension_numbers=(((1,), (1,)), ((), ())),
        preferred_element_type=jnp.float32,
    )


def _build_table(element_embedding, config_weight, electron_config):
    return pl.pallas_call(
        _table_body,
        out_shape=jax.ShapeDtypeStruct((_ZMAX, _NUM_FEATURES), jnp.float32),
    )(element_embedding, electron_config, config_weight)


def _sc_gather(table, z2d):
    zrows, zcols = z2d.shape
    n = zrows * zcols
    b_per_w = n // _NW
    rows_per_w = b_per_w // zcols
    n_chunks = b_per_w // _CHUNK
    mesh = plsc.VectorSubcoreMesh(core_axis_name="c", subcore_axis_name="s")

    @functools.partial(
        pl.kernel,
        mesh=mesh,
        out_type=jax.ShapeDtypeStruct((n, _NUM_FEATURES), jnp.float32),
        scratch_types=[
            pltpu.VMEM_SHARED((_ZMAX, _NUM_FEATURES), jnp.float32),
            pltpu.VMEM((rows_per_w, zcols), jnp.int32),
        ]
        + [pltpu.VMEM((_CHUNK, _NUM_FEATURES), jnp.float32)] * _NBUF
        + [pltpu.SemaphoreType.DMA] * (2 * _NBUF),
    )
    def k(table_hbm, idx_hbm, out_hbm, table_sp, idx_v, *bs):
        bufs, gsems, osems = bs[:_NBUF], bs[_NBUF:2 * _NBUF], bs[2 * _NBUF:]
        wid = lax.axis_index("s") * _NUM_CORES + lax.axis_index("c")
        base = wid * b_per_w

        def start_gather(j, p):
            pltpu.async_copy(
                table_sp.at[idx_v.at[j]],
                bufs[p],
                gsems[p],
            )

        def wait_gather(p):
            pltpu.make_async_copy(
                out_hbm.at[pl.ds(0, _CHUNK)], bufs[p], gsems[p]
            ).wait()

        def wait_scatter(p):
            pltpu.make_async_copy(
                bufs[p], out_hbm.at[pl.ds(0, _CHUNK)], osems[p]
            ).wait()

        @pl.when(lax.axis_index("s") == 0)
        def _():
            pltpu.sync_copy(table_hbm, table_sp)

        pltpu.sync_copy(idx_hbm.at[pl.ds(wid * rows_per_w, rows_per_w)], idx_v)
        plsc.subcore_barrier()

        for j in range(_LOOK):
            start_gather(j, j % _NBUF)

        def body(j, _):
            for p in range(_NBUF):
                @pl.when(j % _NBUF == p)
                def _(p=p):
                    wait_gather(p)
                    pltpu.async_copy(
                        bufs[p],
                        out_hbm.at[pl.ds(base + j * _CHUNK, _CHUNK)],
                        osems[p],
                    )

            @pl.when(j + _LOOK < n_chunks)
            def _():
                for q in range(_NBUF):
                    @pl.when((j + _LOOK) % _NBUF == q)
                    def _(q=q):
                        @pl.when(j + _LOOK >= _NBUF)
                        def _():
                            wait_scatter(q)

                        start_gather(j + _LOOK, q)

            return 0

        lax.fori_loop(0, n_chunks, body, 0)
        for p in range(_NBUF):
            wait_scatter(p)

    return k(table, z2d)


def kernel(Z, element_embedding, config_weight, electron_config):
    table = _build_table(element_embedding, config_weight, electron_config)
    out = _sc_gather(table, Z)
    return out.reshape(Z.shape + (_NUM_FEATURES,))
